# Initial kernel scaffold; baseline (speedup 1.0000x reference)
import jax
import jax.numpy as jnp
from jax.experimental import pallas as pl


def kernel(target, hidden, encoder_outputs, src_tokens, emb, Wih, Whh, bih, bhh, aWh, abh, aWs, abs_, ava, W1, b1, W2, bW2, b2, pWsh, pbsh, pWse, pbse, pWsc, pbsc, pva, pba, paWh, pabh, paWs, pabs, pava):
    raise NotImplementedError("write your pallas kernel here")


# speedup vs baseline: 10.2476x; 10.2476x over previous
import jax
import jax.numpy as jnp
from jax.experimental import pallas as pl
from jax.experimental.pallas import tpu as pltpu

H = 512
V = 32000
S = 128
T = 48
BOS_TOK = 1
NC = 2
VTB = 25
VT = VTB * 128
NVT = V // VT
NVB = V // 128

_CP = getattr(pltpu, "CompilerParams", None) or getattr(pltpu, "TPUCompilerParams")


def _dec_kernel(toks_ref, emb_ref, hid_ref, enc_ref, wiht, whht, awst, ava_r,
                abs_r, bih_r, bhh_r, awht, abh_r,
                h_out, c_out, a_out,
                h_s, whenc_s, xall_s, gsem):
    c = pl.program_id(0)
    t = pl.program_id(1)
    bh = h_s.shape[0]

    @pl.when(t == 0)
    def _init():
        base = c * bh

        def issue(r, carry):
            tt = jax.lax.shift_right_logical(r, 4)
            j = r - tt * bh
            tok = toks_ref[tt, base + j]
            pltpu.make_async_copy(
                emb_ref.at[pl.ds(tok, 1), :], xall_s.at[pl.ds(r, 1), :], gsem
            ).start()
            return carry

        jax.lax.fori_loop(0, T * bh, issue, 0)
        pltpu.make_async_copy(emb_ref.at[pl.ds(0, T * bh), :], xall_s, gsem).wait()
        enc2 = enc_ref[...].reshape(bh * S, H)
        whenc_s[...] = (
            jnp.dot(enc2, awht[...], preferred_element_type=jnp.float32) + abh_r[...]
        ).reshape(bh, S, H)
        h_s[...] = hid_ref[...]

    t16 = pl.multiple_of(t * bh, 8)
    x = xall_s[pl.ds(t16, bh), :]
    h = h_s[...]
    q = jnp.dot(h, awst[...], preferred_element_type=jnp.float32) + abs_r[...]
    tt3 = jnp.tanh(whenc_s[...] + q[:, None, :])
    e = jnp.sum(tt3 * ava_r[...][None, :, :], axis=2)
    m = jnp.max(e, axis=1, keepdims=True)
    p = jnp.exp(e - m)
    a = p / jnp.sum(p, axis=1, keepdims=True)

    enc = enc_ref[...]
    cs = []
    for b in range(bh):
        cs.append(jnp.dot(a[b:b + 1, :], enc[b],
                          preferred_element_type=jnp.float32))
    cvec = jnp.concatenate(cs, axis=0)

    gi = jnp.concatenate([x, cvec], axis=1)
    gx = jnp.dot(gi, wiht[...], preferred_element_type=jnp.float32) + bih_r[...]
    gh = jnp.dot(h, whht[...], preferred_element_type=jnp.float32) + bhh_r[...]
    rg = jax.nn.sigmoid(gx[:, :H] + gh[:, :H])
    zg = jax.nn.sigmoid(gx[:, H:2 * H] + gh[:, H:2 * H])
    ng = jnp.tanh(gx[:, 2 * H:] + rg * gh[:, 2 * H:])
    h_new = (1.0 - zg) * ng + zg * h

    h_s[...] = h_new
    h_out[...] = h_new[None]
    c_out[...] = cvec[None]
    a_out[...] = a[None]


def _tail_kernel(src_ref, w2_ref, h_ref, c_ref, a_ref, enc_ref,
                 pawht, pabh_r, pawst, pabs_r, pava_r,
                 pwsht, pwset, pwsct, psb_r, pva_r,
                 w1t, b1_r, bv_r,
                 out_ref, pg_ref,
                 w2_s, sc3, stage, w2sem, osem):
    ci = pl.program_id(0)
    i = pl.program_id(1)
    bg = ci * pl.num_programs(1) + i

    @pl.when(i == 0)
    def _startw2():
        pltpu.make_async_copy(w2_ref, w2_s, w2sem).start()

    h = h_ref[0]
    cv = c_ref[0]
    av = a_ref[0]
    enc = enc_ref[0]

    penc = jnp.dot(enc, pawht[...], preferred_element_type=jnp.float32) + pabh_r[...]
    q2 = jnp.dot(h, pawst[...], preferred_element_type=jnp.float32) + pabs_r[...]
    pava3 = pava_r[...][None, :, :]
    parts = []
    for sc in range(0, S, 8):
        blk = jnp.tanh(penc[sc:sc + 8][None, :, :] + q2[:, None, :])
        parts.append(jnp.sum(blk * pava3, axis=2))
    e2 = jnp.concatenate(parts, axis=1)
    m2 = jnp.max(e2, axis=1, keepdims=True)
    p2 = jnp.exp(e2 - m2)
    a2 = p2 / jnp.sum(p2, axis=1, keepdims=True)
    c2 = jnp.dot(a2, enc, preferred_element_type=jnp.float32)

    sv = jnp.tanh(
        jnp.dot(h, pwsht[...], preferred_element_type=jnp.float32)
        + jnp.dot(h, pwset[...], preferred_element_type=jnp.float32)
        + jnp.dot(c2, pwsct[...], preferred_element_type=jnp.float32)
        + psb_r[...]
    )
    pg = jax.nn.sigmoid(jnp.sum(sv * pva_r[...], axis=1, keepdims=True))
    pg_ref[...] = pg[None]

    t1 = jnp.tanh(
        jnp.dot(jnp.concatenate([h, cv], axis=1), w1t[...],
                preferred_element_type=jnp.float32) + b1_r[...]
    )
    t1b = t1.astype(jnp.bfloat16)

    @pl.when(i == 0)
    def _waitw2():
        pltpu.make_async_copy(w2_ref, w2_s, w2sem).wait()

    ssum = jnp.zeros((T, 1), jnp.float32)
    for vt in range(NVT):
        vb0 = vt * VTB
        w2blk = w2_s[vb0 * 128:(vb0 + VTB) * 128, :]
        lg = jax.lax.dot_general(
            t1b, w2blk, (((1,), (1,)), ((), ())),
            preferred_element_type=jnp.float32,
        ) + bv_r[:, vb0 * 128:(vb0 + VTB) * 128]
        et = jnp.exp(lg)
        ssum = ssum + jnp.sum(et, axis=1, keepdims=True)
        for j in range(VTB):
            sc3[vb0 + j] = et[:, j * 128:(j + 1) * 128]

    awf = av * ((1.0 - pg) * ssum / pg)
    lane = jax.lax.broadcasted_iota(jnp.int32, (1, 128), 1)
    for s in range(S):
        tok = src_ref[bg, s]
        vb = jax.lax.shift_right_logical(tok, 7)
        ln = jax.lax.bitwise_and(tok, 127)
        addv = jnp.where(lane == ln, awf[:, s:s + 1], 0.0)
        sc3[vb] = sc3[vb] + addv

    inv = pg / ssum
    cps = []
    for vt in range(NVT):
        vb0 = vt * VTB
        tile = jnp.concatenate([sc3[vb0 + j] for j in range(VTB)], axis=1)
        slot = vt & 1
        if vt >= 2:
            cps[vt - 2].wait()
        stage[slot] = tile * inv
        cp = pltpu.make_async_copy(
            stage.at[slot], out_ref.at[bg, :, pl.ds(vb0 * 128, VT)], osem.at[slot]
        )
        cp.start()
        cps.append(cp)
    cps[-2].wait()
    cps[-1].wait()


def kernel(target, hidden, encoder_outputs, src_tokens, emb, Wih, Whh, bih, bhh,
           aWh, abh, aWs, abs_, ava, W1, b1, W2, bW2, b2,
           pWsh, pbsh, pWse, pbse, pWsc, pbsc, pva, pba,
           paWh, pabh, paWs, pabs, pava):
    B = target.shape[0]
    bh = B // NC
    f32 = jnp.float32

    dec_in = jnp.concatenate(
        [jnp.full((B, 1), BOS_TOK, target.dtype), target[:, :-1]], axis=1
    ).T.astype(jnp.int32)
    src_i = src_tokens.astype(jnp.int32)

    row = lambda v: v.reshape(1, -1).astype(f32)
    h_tb, c_tb, a_tb = pl.pallas_call(
        _dec_kernel,
        grid=(NC, T),
        in_specs=[
            pl.BlockSpec(memory_space=pltpu.SMEM),
            pl.BlockSpec(memory_space=pl.ANY),
            pl.BlockSpec((bh, H), lambda c, t: (c, 0)),
            pl.BlockSpec((bh, S, H), lambda c, t: (c, 0, 0)),
            pl.BlockSpec((2 * H, 3 * H), lambda c, t: (0, 0)),
            pl.BlockSpec((H, 3 * H), lambda c, t: (0, 0)),
            pl.BlockSpec((H, H), lambda c, t: (0, 0)),
            pl.BlockSpec((1, H), lambda c, t: (0, 0)),
            pl.BlockSpec((1, H), lambda c, t: (0, 0)),
            pl.BlockSpec((1, 3 * H), lambda c, t: (0, 0)),
            pl.BlockSpec((1, 3 * H), lambda c, t: (0, 0)),
            pl.BlockSpec((H, H), lambda c, t: (0, 0)),
            pl.BlockSpec((1, H), lambda c, t: (0, 0)),
        ],
        out_specs=[
            pl.BlockSpec((1, bh, H), lambda c, t: (t, c, 0)),
            pl.BlockSpec((1, bh, H), lambda c, t: (t, c, 0)),
            pl.BlockSpec((1, bh, S), lambda c, t: (t, c, 0)),
        ],
        out_shape=[
            jax.ShapeDtypeStruct((T, B, H), f32),
            jax.ShapeDtypeStruct((T, B, H), f32),
            jax.ShapeDtypeStruct((T, B, S), f32),
        ],
        scratch_shapes=[
            pltpu.VMEM((bh, H), f32),
            pltpu.VMEM((bh, S, H), f32),
            pltpu.VMEM((T * bh, H), f32),
            pltpu.SemaphoreType.DMA,
        ],
        compiler_params=_CP(
            dimension_semantics=("parallel", "arbitrary"),
            vmem_limit_bytes=100 * 1024 * 1024,
        ),
    )(
        dec_in, emb, hidden, encoder_outputs,
        Wih.T.astype(f32), Whh.T.astype(f32), aWs.T.astype(f32), row(ava),
        row(abs_), row(bih), row(bhh), aWh.T.astype(f32), row(abh),
    )

    h_bt = jnp.transpose(h_tb, (1, 0, 2))
    c_bt = jnp.transpose(c_tb, (1, 0, 2))
    a_bt = jnp.transpose(a_tb, (1, 0, 2))
    w2b = W2.astype(jnp.bfloat16)
    bv = (bW2 + b2).reshape(1, V).astype(f32)
    psb = (pbsh + pbse + pbsc + pba).reshape(1, H).astype(f32)

    out_btv, pg_bt1 = pl.pallas_call(
        _tail_kernel,
        grid=(NC, bh),
        in_specs=[
            pl.BlockSpec(memory_space=pltpu.SMEM),
            pl.BlockSpec(memory_space=pl.ANY),
            pl.BlockSpec((1, T, H), lambda c, i: (c * (B // NC) + i, 0, 0)),
            pl.BlockSpec((1, T, H), lambda c, i: (c * (B // NC) + i, 0, 0)),
            pl.BlockSpec((1, T, S), lambda c, i: (c * (B // NC) + i, 0, 0)),
            pl.BlockSpec((1, S, H), lambda c, i: (c * (B // NC) + i, 0, 0)),
            pl.BlockSpec((H, H), lambda c, i: (0, 0)),
            pl.BlockSpec((1, H), lambda c, i: (0, 0)),
            pl.BlockSpec((H, H), lambda c, i: (0, 0)),
            pl.BlockSpec((1, H), lambda c, i: (0, 0)),
            pl.BlockSpec((1, H), lambda c, i: (0, 0)),
            pl.BlockSpec((H, H), lambda c, i: (0, 0)),
            pl.BlockSpec((H, H), lambda c, i: (0, 0)),
            pl.BlockSpec((H, H), lambda c, i: (0, 0)),
            pl.BlockSpec((1, H), lambda c, i: (0, 0)),
            pl.BlockSpec((1, H), lambda c, i: (0, 0)),
            pl.BlockSpec((2 * H, H), lambda c, i: (0, 0)),
            pl.BlockSpec((1, H), lambda c, i: (0, 0)),
            pl.BlockSpec((1, V), lambda c, i: (0, 0)),
        ],
        out_specs=[
            pl.BlockSpec(memory_space=pl.ANY),
            pl.BlockSpec((1, T, 1), lambda c, i: (c * (B // NC) + i, 0, 0)),
        ],
        out_shape=[
            jax.ShapeDtypeStruct((B, T, V), f32),
            jax.ShapeDtypeStruct((B, T, 1), f32),
        ],
        scratch_shapes=[
            pltpu.VMEM((V, H), jnp.bfloat16),
            pltpu.VMEM((NVB, T, 128), f32),
            pltpu.VMEM((2, T, VT), f32),
            pltpu.SemaphoreType.DMA,
            pltpu.SemaphoreType.DMA((2,)),
        ],
        compiler_params=_CP(
            dimension_semantics=("parallel", "arbitrary"),
            vmem_limit_bytes=100 * 1024 * 1024,
        ),
    )(
        src_i, w2b, h_bt, c_bt, a_bt, encoder_outputs,
        paWh.T.astype(f32), row(pabh), paWs.T.astype(f32), row(pabs), row(pava),
        pWsh.T.astype(f32), pWse.T.astype(f32), pWsc.T.astype(f32), psb, row(pva),
        W1.T.astype(f32), row(b1), bv,
    )

    return out_btv, h_bt[:, -1, :], a_bt[:, -1, :], pg_bt1[:, -1, :]
